# baseline (speedup 1.0000x reference)
import functools

import jax
import jax.numpy as jnp
from jax import lax
from jax.experimental import pallas as pl
from jax.experimental.pallas import tpu as pltpu
from jax.experimental.pallas import tpu_sc as plsc

_L = 16


def _build_sc_kernel(B, Dx, Dy, NC, NS):
    NW = NC * NS
    b_per_w = B // NW
    n_groups = b_per_w // _L
    D = Dx + Dy
    mesh = plsc.VectorSubcoreMesh(core_axis_name="c", subcore_axis_name="s")

    @functools.partial(
        pl.kernel,
        mesh=mesh,
        compiler_params=pltpu.CompilerParams(
            needs_layout_passes=False, use_tc_tiling_on_sc=True),
        out_type=jax.ShapeDtypeStruct((B, D), jnp.float32),
        scratch_types=[
            pltpu.VMEM((b_per_w,), jnp.int32),
            pltpu.VMEM((b_per_w,), jnp.int32),
            pltpu.VMEM((2, _L, Dx), jnp.float32),
            pltpu.VMEM((2, _L, Dy), jnp.float32),
            pltpu.VMEM((8 * _L, D), jnp.float32),
            pltpu.SemaphoreType.DMA,
            pltpu.SemaphoreType.DMA,
        ],
    )
    def k(xi_hbm, yi_hbm, wx_hbm, wy_hbm, out_hbm,
          xi_v, yi_v, bx_v, by_v, cat_v, sem0, sem1):
        wx3 = wx_hbm.reshape(wx_hbm.shape[0] // 8, 8, Dx)
        wy3 = wy_hbm.reshape(wy_hbm.shape[0] // 8, 8, Dy)
        wid = lax.axis_index("s") * NC + lax.axis_index("c")
        base = wid * b_per_w
        pltpu.sync_copy(xi_hbm.at[pl.ds(base, b_per_w)], xi_v)
        pltpu.sync_copy(yi_hbm.at[pl.ds(base, b_per_w)], yi_v)

        def fire(g, par, sem):
            idxv = xi_v[pl.ds(g * _L, _L)]
            idyv = yi_v[pl.ds(g * _L, _L)]
            for kk in range(_L):
                i = idxv[kk]
                pltpu.async_copy(
                    wx3.at[lax.shift_right_logical(i, 3), lax.bitwise_and(i, 7)],
                    bx_v.at[par, kk], sem)
            for kk in range(_L):
                i = idyv[kk]
                pltpu.async_copy(
                    wy3.at[lax.shift_right_logical(i, 3), lax.bitwise_and(i, 7)],
                    by_v.at[par, kk], sem)

        def wait(par, sem):
            pltpu.make_async_copy(
                wx3.at[pl.ds(0, 2), pl.ds(0, 8)].reshape(_L, Dx),
                bx_v.at[par], sem).wait()
            pltpu.make_async_copy(
                wy3.at[pl.ds(0, 2), pl.ds(0, 8)].reshape(_L, Dy),
                by_v.at[par], sem).wait()

        def extract(g, par, row0):
            for kk in range(_L):
                for c0 in range(0, Dx, _L):
                    cat_v[row0 + kk, pl.ds(c0, _L)] = bx_v[par, kk, pl.ds(c0, _L)]
                for c0 in range(0, Dy, _L):
                    cat_v[row0 + kk, pl.ds(Dx + c0, _L)] = by_v[par, kk, pl.ds(c0, _L)]

        def window(w, carry):
            g0 = 8 * w
            fire(g0, 0, sem0)
            fire(g0 + 1, 1, sem1)
            for u in range(4):
                wait(0, sem0)
                extract(g0 + 2 * u, 0, 2 * u * _L)
                if u < 3:
                    fire(g0 + 2 * u + 2, 0, sem0)
                wait(1, sem1)
                extract(g0 + 2 * u + 1, 1, (2 * u + 1) * _L)
                if u < 3:
                    fire(g0 + 2 * u + 3, 1, sem1)
            pltpu.sync_copy(
                cat_v, out_hbm.at[pl.ds(base + w * 8 * _L, 8 * _L)])
            return carry
        lax.fori_loop(0, n_groups // 8, window, 0)

    return k


def kernel(x, Wx, Wy):
    B = x.shape[0]
    Dx = Wx.shape[1]
    Dy = Wy.shape[1]
    info = plsc.get_sparse_core_info()
    k = _build_sc_kernel(B, Dx, Dy, info.num_cores, info.num_subcores)
    x32 = x.astype(jnp.int32)
    return k(x32[:, 0], x32[:, 1], Wx, Wy)
